# baseline (speedup 1.0000x reference)
import jax
import jax.numpy as jnp
from jax.experimental import pallas as pl
from jax.experimental.pallas import tpu as pltpu

_SIGMA_MIN = 0.01
_SIGMA_MAX = 50.0
_HIGHEST = jax.lax.Precision.HIGHEST


def _body_large(scal_ref, xt_ref, x0_ref, ll_ref, lx_ref, lll_ref, out_ref):
    t_scale = scal_ref[0]
    log_t = scal_ref[1]
    half_inv_t2 = scal_ref[2]
    nb = lx_ref.shape[0]
    d = xt_ref.shape[-1]
    ones_bf = jnp.ones((1, d), dtype=jnp.bfloat16)

    for i in range(nb):
        xt = xt_ref[i]
        lx = lx_ref[i]
        lll = lll_ref[i]
        x0 = x0_ref[i]
        llk = ll_ref[i]

        y = lx - xt
        q = y * y
        qh = q.astype(jnp.bfloat16)
        ql = (q - qh.astype(jnp.float32)).astype(jnp.bfloat16)
        n2 = (jax.lax.dot_general(ones_bf, qh, (((1,), (1,)), ((), ())),
                                  preferred_element_type=jnp.float32)
              + jax.lax.dot_general(ones_bf, ql, (((1,), (1,)), ((), ())),
                                    preferred_element_type=jnp.float32))
        u_live = (lll - half_inv_t2 * n2) - log_t

        yk = x0 - xt
        u_k = llk

        m = jnp.maximum(jnp.max(u_live), jnp.max(u_k))
        el = jnp.exp(u_live - m)
        ek = jnp.exp(u_k - m)
        s_tot = jnp.sum(el) + jnp.sum(ek)

        ws = (jax.lax.dot_general(el, y, (((1,), (0,)), ((), ())),
                                  preferred_element_type=jnp.float32)
              + jax.lax.dot_general(ek, yk, (((1,), (0,)), ((), ())),
                                    preferred_element_type=jnp.float32))
        out_ref[i] = ws / (s_tot * t_scale)


def _body_small(scal_ref, xt_ref, x0_ref, ll_ref, s1_ref, s2_ref, out_ref):
    t_scale = scal_ref[0]
    log_t = scal_ref[1]
    xt = xt_ref[...]
    ll = ll_ref[...]
    s1 = s1_ref[...]
    s2full = s2_ref[...]
    k_new = ll.shape[1]
    s2k = s2full[:, s2full.shape[1] - k_new:]

    u_k = (((ll + s1) + log_t) - s2k) - log_t
    m = jnp.max(u_k, axis=1, keepdims=True)
    ek = jnp.exp(u_k - m)
    s_tot = jnp.sum(ek, axis=1, keepdims=True)

    ws = ek[:, 0:1] * x0_ref[:, 0, :]
    for k in range(1, k_new):
        ws = ws + ek[:, k:k + 1] * x0_ref[:, k, :]
    out_ref[...] = (ws - s_tot * xt) / (s_tot * t_scale)


def kernel(t, xt, x0, ll, live_x0, live_ll):
    B, D = xt.shape
    K = x0.shape[1]
    L = live_x0.shape[1]

    t_scale = _SIGMA_MIN * (_SIGMA_MAX / _SIGMA_MIN) ** t[0]
    log_t = jnp.log(t_scale)
    half_inv_t2 = 0.5 / (t_scale * t_scale)
    scal = jnp.stack([t_scale, log_t, half_inv_t2]).astype(jnp.float32)

    def large_path(scal, xt, x0, ll, live_x0, live_ll):
        nb = 8
        xt3 = xt.reshape(B, 1, D)
        ll3 = ll.reshape(B, 1, K)
        lll3 = live_ll.reshape(B, 1, L)
        out = pl.pallas_call(
            _body_large,
            grid=(B // nb,),
            in_specs=[
                pl.BlockSpec(memory_space=pltpu.SMEM),
                pl.BlockSpec((nb, 1, D), lambda b: (b, 0, 0)),
                pl.BlockSpec((nb, K, D), lambda b: (b, 0, 0)),
                pl.BlockSpec((nb, 1, K), lambda b: (b, 0, 0)),
                pl.BlockSpec((nb, L, D), lambda b: (b, 0, 0)),
                pl.BlockSpec((nb, 1, L), lambda b: (b, 0, 0)),
            ],
            out_specs=pl.BlockSpec((nb, 1, D), lambda b: (b, 0, 0)),
            out_shape=jax.ShapeDtypeStruct((B, 1, D), jnp.float32),
        )(scal, xt3, x0, ll3, live_x0, lll3)
        return out.reshape(B, D)

    def small_path(scal, xt, x0, ll, live_x0, live_ll):
        diff_new = x0 - xt[:, None, :]
        s1 = 0.5 * jnp.sum(diff_new ** 2 / t_scale ** 2, axis=2)
        current_x0 = jnp.concatenate([live_x0, x0], axis=1)
        diff_cat = current_x0 - xt[:, None, :]
        s2full = 0.5 * jnp.sum(diff_cat ** 2 / t_scale ** 2, axis=2)
        out = pl.pallas_call(
            _body_small,
            in_specs=[
                pl.BlockSpec(memory_space=pltpu.SMEM),
                pl.BlockSpec((B, D), lambda: (0, 0)),
                pl.BlockSpec((B, K, D), lambda: (0, 0, 0)),
                pl.BlockSpec((B, K), lambda: (0, 0)),
                pl.BlockSpec((B, K), lambda: (0, 0)),
                pl.BlockSpec((B, L + K), lambda: (0, 0)),
            ],
            out_specs=pl.BlockSpec((B, D), lambda: (0, 0)),
            out_shape=jax.ShapeDtypeStruct((B, D), jnp.float32),
        )(scal, xt, x0, ll.reshape(B, K), s1, s2full)
        return out

    return jax.lax.cond(t_scale < 0.5, small_path, large_path,
                        scal, xt, x0, ll, live_x0, live_ll)


# speedup vs baseline: 1.3495x; 1.3495x over previous
import functools

import jax
import jax.numpy as jnp
from jax.experimental import pallas as pl
from jax.experimental.pallas import tpu as pltpu

_SIGMA_MIN = 0.01
_SIGMA_MAX = 50.0
_HIGHEST = jax.lax.Precision.HIGHEST


def _body_large(split_n2, scal_ref, xt_ref, x0_ref, ll_ref, lx_ref, lll_ref,
                out_ref):
    t_scale = scal_ref[0]
    log_t = scal_ref[1]
    half_inv_t2 = scal_ref[2]
    nb = lx_ref.shape[0]
    d = xt_ref.shape[-1]
    ones_bf = jnp.ones((1, d), dtype=jnp.bfloat16)

    for i in range(nb):
        xt = xt_ref[i]
        lx = lx_ref[i]
        lll = lll_ref[i]
        x0 = x0_ref[i]
        llk = ll_ref[i]

        y = lx - xt
        q = y * y
        qh = q.astype(jnp.bfloat16)
        n2 = jax.lax.dot_general(ones_bf, qh, (((1,), (1,)), ((), ())),
                                 preferred_element_type=jnp.float32)
        if split_n2:
            ql = (q - qh.astype(jnp.float32)).astype(jnp.bfloat16)
            n2 = n2 + jax.lax.dot_general(ones_bf, ql,
                                          (((1,), (1,)), ((), ())),
                                          preferred_element_type=jnp.float32)
        u_live = (lll - half_inv_t2 * n2) - log_t

        yk = x0 - xt
        u_k = llk

        m = jnp.maximum(jnp.max(u_live), jnp.max(u_k))
        el = jnp.exp(u_live - m)
        ek = jnp.exp(u_k - m)
        s_tot = jnp.sum(el) + jnp.sum(ek)

        ws = (jax.lax.dot_general(el, y, (((1,), (0,)), ((), ())),
                                  preferred_element_type=jnp.float32)
              + jax.lax.dot_general(ek, yk, (((1,), (0,)), ((), ())),
                                    preferred_element_type=jnp.float32))
        out_ref[i] = ws / (s_tot * t_scale)


def _body_small(scal_ref, xt_ref, x0_ref, ll_ref, s1_ref, s2_ref, out_ref):
    t_scale = scal_ref[0]
    log_t = scal_ref[1]
    xt = xt_ref[...]
    ll = ll_ref[...]
    s1 = s1_ref[...]
    s2full = s2_ref[...]
    k_new = ll.shape[1]
    s2k = s2full[:, s2full.shape[1] - k_new:]

    u_k = (((ll + s1) + log_t) - s2k) - log_t
    m = jnp.max(u_k, axis=1, keepdims=True)
    ek = jnp.exp(u_k - m)
    s_tot = jnp.sum(ek, axis=1, keepdims=True)

    ws = ek[:, 0:1] * x0_ref[:, 0, :]
    for k in range(1, k_new):
        ws = ws + ek[:, k:k + 1] * x0_ref[:, k, :]
    out_ref[...] = (ws - s_tot * xt) / (s_tot * t_scale)


def kernel(t, xt, x0, ll, live_x0, live_ll):
    B, D = xt.shape
    K = x0.shape[1]
    L = live_x0.shape[1]

    t_scale = _SIGMA_MIN * (_SIGMA_MAX / _SIGMA_MIN) ** t[0]
    log_t = jnp.log(t_scale)
    half_inv_t2 = 0.5 / (t_scale * t_scale)
    scal = jnp.stack([t_scale, log_t, half_inv_t2]).astype(jnp.float32)

    def _large_path(split_n2, scal, xt, x0, ll, live_x0, live_ll):
        nb = 4
        xt3 = xt.reshape(B, 1, D)
        ll3 = ll.reshape(B, 1, K)
        lll3 = live_ll.reshape(B, 1, L)
        out = pl.pallas_call(
            functools.partial(_body_large, split_n2),
            grid=(B // nb,),
            in_specs=[
                pl.BlockSpec(memory_space=pltpu.SMEM),
                pl.BlockSpec((nb, 1, D), lambda b: (b, 0, 0)),
                pl.BlockSpec((nb, K, D), lambda b: (b, 0, 0)),
                pl.BlockSpec((nb, 1, K), lambda b: (b, 0, 0)),
                pl.BlockSpec((nb, L, D), lambda b: (b, 0, 0)),
                pl.BlockSpec((nb, 1, L), lambda b: (b, 0, 0)),
            ],
            out_specs=pl.BlockSpec((nb, 1, D), lambda b: (b, 0, 0)),
            out_shape=jax.ShapeDtypeStruct((B, 1, D), jnp.float32),
        )(scal, xt3, x0, ll3, live_x0, lll3)
        return out.reshape(B, D)

    def small_path(scal, xt, x0, ll, live_x0, live_ll):
        diff_new = x0 - xt[:, None, :]
        s1 = 0.5 * jnp.sum(diff_new ** 2 / t_scale ** 2, axis=2)
        current_x0 = jnp.concatenate([live_x0, x0], axis=1)
        diff_cat = current_x0 - xt[:, None, :]
        s2full = 0.5 * jnp.sum(diff_cat ** 2 / t_scale ** 2, axis=2)
        out = pl.pallas_call(
            _body_small,
            in_specs=[
                pl.BlockSpec(memory_space=pltpu.SMEM),
                pl.BlockSpec((B, D), lambda: (0, 0)),
                pl.BlockSpec((B, K, D), lambda: (0, 0, 0)),
                pl.BlockSpec((B, K), lambda: (0, 0)),
                pl.BlockSpec((B, K), lambda: (0, 0)),
                pl.BlockSpec((B, L + K), lambda: (0, 0)),
            ],
            out_specs=pl.BlockSpec((B, D), lambda: (0, 0)),
            out_shape=jax.ShapeDtypeStruct((B, D), jnp.float32),
        )(scal, xt, x0, ll.reshape(B, K), s1, s2full)
        return out

    def large_path(scal, xt, x0, ll, live_x0, live_ll):
        return jax.lax.cond(t_scale < 4.0,
                            functools.partial(_large_path, True),
                            functools.partial(_large_path, False),
                            scal, xt, x0, ll, live_x0, live_ll)

    return jax.lax.cond(t_scale < 0.5, small_path, large_path,
                        scal, xt, x0, ll, live_x0, live_ll)


# speedup vs baseline: 1.3507x; 1.0009x over previous
import functools

import jax
import jax.numpy as jnp
from jax.experimental import pallas as pl
from jax.experimental.pallas import tpu as pltpu

_SIGMA_MIN = 0.01
_SIGMA_MAX = 50.0


def _body_large(split_n2, scal_ref, xt_ref, x0_ref, ll_ref, lx_ref, lll_ref,
                out_ref):
    t_scale = scal_ref[0]
    log_t = scal_ref[1]
    half_inv_t2 = scal_ref[2]
    nb = lx_ref.shape[0]
    d = xt_ref.shape[-1]
    ones_bf = jnp.ones((1, d), dtype=jnp.bfloat16)

    for i in range(nb):
        xt = xt_ref[i]
        lx = lx_ref[i]
        lll = lll_ref[i]
        x0 = x0_ref[i]
        llk = ll_ref[i]

        y = lx - xt
        q = y * y
        qh = q.astype(jnp.bfloat16)
        n2 = jax.lax.dot_general(ones_bf, qh, (((1,), (1,)), ((), ())),
                                 preferred_element_type=jnp.float32)
        if split_n2:
            ql = (q - qh.astype(jnp.float32)).astype(jnp.bfloat16)
            n2 = n2 + jax.lax.dot_general(ones_bf, ql,
                                          (((1,), (1,)), ((), ())),
                                          preferred_element_type=jnp.float32)
        u_live = (lll - half_inv_t2 * n2) - log_t

        yk = x0 - xt
        u_k = llk

        m = jnp.maximum(jnp.max(u_live), jnp.max(u_k))
        el = jnp.exp(u_live - m)
        ek = jnp.exp(u_k - m)
        s_tot = jnp.sum(el) + jnp.sum(ek)

        ws = (jax.lax.dot_general(el, y, (((1,), (0,)), ((), ())),
                                  preferred_element_type=jnp.float32)
              + jax.lax.dot_general(ek, yk, (((1,), (0,)), ((), ())),
                                    preferred_element_type=jnp.float32))
        out_ref[i] = ws / (s_tot * t_scale)


def _body_small(scal_ref, xt_ref, x0_ref, ll_ref, s1_ref, s2_ref, out_ref):
    t_scale = scal_ref[0]
    log_t = scal_ref[1]
    xt = xt_ref[...]
    ll = ll_ref[...]
    s1 = s1_ref[...]
    s2full = s2_ref[...]
    k_new = ll.shape[1]
    s2k = s2full[:, s2full.shape[1] - k_new:]

    u_k = (((ll + s1) + log_t) - s2k) - log_t
    m = jnp.max(u_k, axis=1, keepdims=True)
    ek = jnp.exp(u_k - m)
    s_tot = jnp.sum(ek, axis=1, keepdims=True)

    ws = ek[:, 0:1] * x0_ref[:, 0, :]
    for k in range(1, k_new):
        ws = ws + ek[:, k:k + 1] * x0_ref[:, k, :]
    out_ref[...] = (ws - s_tot * xt) / (s_tot * t_scale)


def kernel(t, xt, x0, ll, live_x0, live_ll):
    B, D = xt.shape
    K = x0.shape[1]
    L = live_x0.shape[1]

    t_scale = _SIGMA_MIN * (_SIGMA_MAX / _SIGMA_MIN) ** t[0]
    log_t = jnp.log(t_scale)
    half_inv_t2 = 0.5 / (t_scale * t_scale)
    scal = jnp.stack([t_scale, log_t, half_inv_t2]).astype(jnp.float32)

    def _large_path(split_n2, scal, xt, x0, ll, live_x0, live_ll):
        nb = 4
        xt3 = xt.reshape(B, 1, D)
        ll3 = ll.reshape(B, 1, K)
        lll3 = live_ll.reshape(B, 1, L)
        out = pl.pallas_call(
            functools.partial(_body_large, split_n2),
            grid=(B // nb,),
            in_specs=[
                pl.BlockSpec(memory_space=pltpu.SMEM),
                pl.BlockSpec((nb, 1, D), lambda b: (b, 0, 0)),
                pl.BlockSpec((nb, K, D), lambda b: (b, 0, 0)),
                pl.BlockSpec((nb, 1, K), lambda b: (b, 0, 0)),
                pl.BlockSpec((nb, L, D), lambda b: (b, 0, 0)),
                pl.BlockSpec((nb, 1, L), lambda b: (b, 0, 0)),
            ],
            out_specs=pl.BlockSpec((nb, 1, D), lambda b: (b, 0, 0)),
            out_shape=jax.ShapeDtypeStruct((B, 1, D), jnp.float32),
        )(scal, xt3, x0, ll3, live_x0, lll3)
        return out.reshape(B, D)

    def small_path(scal, xt, x0, ll, live_x0, live_ll):
        diff_new = x0 - xt[:, None, :]
        s1 = 0.5 * jnp.sum(diff_new ** 2 / t_scale ** 2, axis=2)
        current_x0 = jnp.concatenate([live_x0, x0], axis=1)
        diff_cat = current_x0 - xt[:, None, :]
        s2full = 0.5 * jnp.sum(diff_cat ** 2 / t_scale ** 2, axis=2)
        out = pl.pallas_call(
            _body_small,
            in_specs=[
                pl.BlockSpec(memory_space=pltpu.SMEM),
                pl.BlockSpec((B, D), lambda: (0, 0)),
                pl.BlockSpec((B, K, D), lambda: (0, 0, 0)),
                pl.BlockSpec((B, K), lambda: (0, 0)),
                pl.BlockSpec((B, K), lambda: (0, 0)),
                pl.BlockSpec((B, L + K), lambda: (0, 0)),
            ],
            out_specs=pl.BlockSpec((B, D), lambda: (0, 0)),
            out_shape=jax.ShapeDtypeStruct((B, D), jnp.float32),
        )(scal, xt, x0, ll.reshape(B, K), s1, s2full)
        return out

    def large_path(scal, xt, x0, ll, live_x0, live_ll):
        return jax.lax.cond(t_scale < 4.0,
                            functools.partial(_large_path, True),
                            functools.partial(_large_path, False),
                            scal, xt, x0, ll, live_x0, live_ll)

    return jax.lax.cond(t_scale < 0.5, small_path, large_path,
                        scal, xt, x0, ll, live_x0, live_ll)
